# Initial kernel scaffold; baseline (speedup 1.0000x reference)
import jax
import jax.numpy as jnp
from jax.experimental import pallas as pl


def kernel(x, embedding, Wt, bt, Wg, bg, We, be, bias_lang, bias0, log_scale):
    raise NotImplementedError("write your pallas kernel here")


# speedup vs baseline: 1.4054x; 1.4054x over previous
import jax
import jax.numpy as jnp
from jax.experimental import pallas as pl
from jax.experimental.pallas import tpu as pltpu

_DL = 768
_DO = 256
_E = 8
_WCAT = _DO + _E * _DO + 128
_GCOL = _DO + _E * _DO
_BCOL = _GCOL + _E


def _body(x_ref, emb_ref, wcat_ref, bvec_ref, out_ref):
    emb = emb_ref[0]
    nrm = jnp.sqrt(jnp.sum(emb * emb, axis=1, keepdims=True))
    en = emb / jnp.maximum(nrm, 1e-12)
    en_bf = en.astype(jnp.bfloat16)

    y = jax.lax.dot_general(
        en_bf, wcat_ref[...],
        dimension_numbers=(((1,), (0,)), ((), ())),
        preferred_element_type=jnp.float32,
    ) + bvec_ref[...]

    gate = y[:, _GCOL:_GCOL + _E]
    gw = jax.nn.softmax(gate, axis=1)
    iota = jax.lax.broadcasted_iota(jnp.int32, gw.shape, 1)
    i1 = jnp.argmax(gw, axis=1)[:, None]
    v1 = jnp.max(gw, axis=1, keepdims=True)
    gw2 = jnp.where(iota == i1, -1.0, gw)
    i2 = jnp.argmax(gw2, axis=1)[:, None]
    v2 = jnp.max(gw2, axis=1, keepdims=True)
    wmask = jnp.where(iota == i1, v1, 0.0) + jnp.where(iota == i2, v2, 0.0)

    tok = y[:, :_DO]
    for e in range(_E):
        tok = tok + wmask[:, e:e + 1] * y[:, _DO + e * _DO:_DO + (e + 1) * _DO]

    bias_tok = y[:, _BCOL:_BCOL + 1]

    logit = jax.lax.dot_general(
        x_ref[0], tok.astype(jnp.bfloat16),
        dimension_numbers=(((1,), (1,)), ((), ())),
        preferred_element_type=jnp.float32,
    ) + bias_tok.T
    out_ref[0] = jnp.clip(logit, -50000.0, 50000.0)


def kernel(x, embedding, Wt, bt, Wg, bg, We, be, bias_lang, bias0, log_scale):
    B, A, DO = x.shape
    L = embedding.shape[1]
    DL = embedding.shape[2]
    E = Wg.shape[0]

    inv = jnp.exp(-log_scale[0])
    half_inv = 0.5 * inv

    wcat = jnp.zeros((DL, _WCAT), dtype=jnp.float32)
    wcat = wcat.at[:, :DO].set(Wt.T * half_inv)
    wcat = wcat.at[:, DO:DO + E * DO].set(
        jnp.transpose(We, (2, 0, 1)).reshape(DL, E * DO) * half_inv)
    wcat = wcat.at[:, _GCOL:_GCOL + E].set(Wg.T)
    wcat = wcat.at[:, _BCOL].set(bias_lang)
    wcat = wcat.astype(jnp.bfloat16)

    bvec = jnp.zeros((1, _WCAT), dtype=jnp.float32)
    bvec = bvec.at[0, :DO].set(bt * half_inv)
    bvec = bvec.at[0, DO:DO + E * DO].set(be.reshape(E * DO) * half_inv)
    bvec = bvec.at[0, _GCOL:_GCOL + E].set(bg)
    bvec = bvec.at[0, _BCOL].set(bias0[0])

    x_bf = x.astype(jnp.bfloat16)

    return pl.pallas_call(
        _body,
        grid=(B,),
        in_specs=[
            pl.BlockSpec((1, A, DO), lambda b: (b, 0, 0)),
            pl.BlockSpec((1, L, DL), lambda b: (b, 0, 0)),
            pl.BlockSpec((DL, _WCAT), lambda b: (0, 0)),
            pl.BlockSpec((1, _WCAT), lambda b: (0, 0)),
        ],
        out_specs=pl.BlockSpec((1, A, L), lambda b: (b, 0, 0)),
        out_shape=jax.ShapeDtypeStruct((B, A, L), jnp.float32),
    )(x_bf, embedding, wcat, bvec)
